# baseline (speedup 1.0000x reference)
import functools

import jax
import jax.numpy as jnp
from jax import lax
from jax.experimental import pallas as pl
from jax.experimental.pallas import tpu as pltpu
from jax.experimental.pallas import tpu_sc as plsc

B = 16384
E = 32
H1 = E // 2
H2 = E // 4
H3 = E // 8

NC, NS = 2, 16
NW = NC * NS
BPW = B // NW
CHUNK = 128
NCH = BPW // CHUNK

@functools.cache
def _build_sc_gather4():
    mesh = plsc.VectorSubcoreMesh(core_axis_name="c", subcore_axis_name="s")

    @functools.partial(
        pl.kernel,
        mesh=mesh,
        out_type=[jax.ShapeDtypeStruct((B, E), jnp.float32)] * 4,
        scratch_types=[
            pltpu.VMEM((NCH, CHUNK), jnp.int32),
            pltpu.VMEM((NCH, CHUNK), jnp.int32),
            pltpu.VMEM((BPW, E), jnp.float32),
            pltpu.VMEM((BPW, E), jnp.float32),
            pltpu.VMEM((BPW, E), jnp.float32),
            pltpu.VMEM((BPW, E), jnp.float32),
            pltpu.SemaphoreType.DMA,
        ],
        compiler_params=pltpu.CompilerParams(use_tc_tiling_on_sc=False),
    )
    def _sc_gather4(user_h, item_h, t_gu, t_gi, t_mu, t_mi,
                    o_gu, o_gi, o_mu, o_mi,
                    idx_u, idx_i, r_gu, r_gi, r_mu, r_mi, sem):
        wid = lax.axis_index("s") * NC + lax.axis_index("c")
        base = wid * BPW
        for c in range(NCH):
            pltpu.sync_copy(user_h.at[pl.ds(base + c * CHUNK, CHUNK)],
                            idx_u.at[c])
            pltpu.sync_copy(item_h.at[pl.ds(base + c * CHUNK, CHUNK)],
                            idx_i.at[c])
        copies = []
        for c in range(NCH):
            sl = pl.ds(c * CHUNK, CHUNK)
            copies.append(pltpu.async_copy(t_gu.at[idx_u.at[c]], r_gu.at[sl], sem))
            copies.append(pltpu.async_copy(t_gi.at[idx_i.at[c]], r_gi.at[sl], sem))
            copies.append(pltpu.async_copy(t_mu.at[idx_u.at[c]], r_mu.at[sl], sem))
            copies.append(pltpu.async_copy(t_mi.at[idx_i.at[c]], r_mi.at[sl], sem))
        for cp in copies:
            cp.wait()
        pltpu.sync_copy(r_gu, o_gu.at[pl.ds(base, BPW)])
        pltpu.sync_copy(r_gi, o_gi.at[pl.ds(base, BPW)])
        pltpu.sync_copy(r_mu, o_mu.at[pl.ds(base, BPW)])
        pltpu.sync_copy(r_mi, o_mi.at[pl.ds(base, BPW)])

    return _sc_gather4


BLK = 2048


def _tc_body(gu, gi, mu, mi, w1a, w1b, b1, w2, b2, w3, b3, wog, wom, bo, out):
    h = jnp.dot(mu[...], w1a[...], preferred_element_type=jnp.float32)
    h = h + jnp.dot(mi[...], w1b[...], preferred_element_type=jnp.float32)
    h = jnp.maximum(h + b1[...], 0.0)
    h = jnp.maximum(
        jnp.dot(h, w2[...], preferred_element_type=jnp.float32) + b2[...], 0.0)
    pm = jnp.dot(h, w3[...], preferred_element_type=jnp.float32) + b3[...]
    pg = gu[...] * gi[...]
    logit = (jnp.sum(pg * wog[...], axis=1, keepdims=True)
             + jnp.sum(pm * wom[...], axis=1, keepdims=True) + bo[...])
    out[...] = jax.nn.sigmoid(logit)


def _tc_dense(gu, gi, mu, mi, w1a, w1b, b1, w2, b2, w3, b3, wog, wom, bo):
    full = lambda shape: pl.BlockSpec(shape, lambda i: (0, 0))
    return pl.pallas_call(
        _tc_body,
        grid=(B // BLK,),
        in_specs=[
            pl.BlockSpec((BLK, E), lambda i: (i, 0)),
            pl.BlockSpec((BLK, E), lambda i: (i, 0)),
            pl.BlockSpec((BLK, E), lambda i: (i, 0)),
            pl.BlockSpec((BLK, E), lambda i: (i, 0)),
            full((E, H1)), full((E, H1)), full((1, H1)),
            full((H1, H2)), full((1, H2)),
            full((H2, H3)), full((1, H3)),
            full((1, E)), full((1, H3)), full((1, 1)),
        ],
        out_specs=pl.BlockSpec((BLK, 1), lambda i: (i, 0)),
        out_shape=jax.ShapeDtypeStruct((B, 1), jnp.float32),
    )(gu, gi, mu, mi, w1a, w1b, b1, w2, b2, w3, b3, wog, wom, bo)


def kernel(user, item, gmf_user_w, gmf_item_w, mlp_user_w, mlp_item_w,
           W1, b1, W2, b2, W3, b3, Wo, bo):
    gu, gi, mu, mi = _build_sc_gather4()(user, item, gmf_user_w, gmf_item_w,
                                         mlp_user_w, mlp_item_w)
    w1a, w1b = W1[:E], W1[E:]
    wog = Wo[:E].reshape(1, E)
    wom = Wo[E:].reshape(1, H3)
    return _tc_dense(gu, gi, mu, mi, w1a, w1b,
                     b1.reshape(1, H1), W2, b2.reshape(1, H2),
                     W3, b3.reshape(1, H3), wog, wom, bo.reshape(1, 1))


# speedup vs baseline: 1.4146x; 1.4146x over previous
import functools

import jax
import jax.numpy as jnp
from jax import lax
from jax.experimental import pallas as pl
from jax.experimental.pallas import tpu as pltpu
from jax.experimental.pallas import tpu_sc as plsc

B = 16384
E = 32
H1 = E // 2
H2 = E // 4
H3 = E // 8

NC, NS = 2, 16
NW = NC * NS
BPW = B // NW
CHUNK = 128
NCH = BPW // CHUNK

@functools.cache
def _build_sc_gather4():
    mesh = plsc.VectorSubcoreMesh(core_axis_name="c", subcore_axis_name="s")

    @functools.partial(
        pl.kernel,
        mesh=mesh,
        out_type=[jax.ShapeDtypeStruct((B, E), jnp.float32)] * 4,
        scratch_types=[
            pltpu.VMEM((BPW,), jnp.int32),
            pltpu.VMEM((BPW,), jnp.int32),
            pltpu.VMEM((CHUNK, E), jnp.float32),
            pltpu.VMEM((CHUNK, E), jnp.float32),
            pltpu.VMEM((CHUNK, E), jnp.float32),
            pltpu.VMEM((CHUNK, E), jnp.float32),
            pltpu.SemaphoreType.DMA,
        ],
    )
    def _sc_gather4(user_h, item_h, t_gu, t_gi, t_mu, t_mi,
                    o_gu, o_gi, o_mu, o_mi,
                    idx_u, idx_i, r_gu, r_gi, r_mu, r_mi, sem):
        wid = lax.axis_index("s") * NC + lax.axis_index("c")
        base = wid * BPW
        pltpu.sync_copy(user_h.at[pl.ds(base, BPW)], idx_u)
        pltpu.sync_copy(item_h.at[pl.ds(base, BPW)], idx_i)

        for c in range(NCH):
            def group(g, carry):
                uvec = idx_u[pl.ds(c * CHUNK + g * 16, 16)]
                ivec = idx_i[pl.ds(c * CHUNK + g * 16, 16)]
                for l in range(16):
                    j = g * 16 + l
                    u = uvec[l]
                    i = ivec[l]
                    pltpu.async_copy(t_gu.at[u], r_gu.at[j], sem)
                    pltpu.async_copy(t_gi.at[i], r_gi.at[j], sem)
                    pltpu.async_copy(t_mu.at[u], r_mu.at[j], sem)
                    pltpu.async_copy(t_mi.at[i], r_mi.at[j], sem)
                return carry

            lax.fori_loop(0, CHUNK // 16, group, 0)
            pltpu.make_async_copy(t_gu.at[pl.ds(0, CHUNK)], r_gu, sem).wait()
            pltpu.make_async_copy(t_gi.at[pl.ds(0, CHUNK)], r_gi, sem).wait()
            pltpu.make_async_copy(t_mu.at[pl.ds(0, CHUNK)], r_mu, sem).wait()
            pltpu.make_async_copy(t_mi.at[pl.ds(0, CHUNK)], r_mi, sem).wait()
            dst = pl.ds(base + c * CHUNK, CHUNK)
            pltpu.sync_copy(r_gu, o_gu.at[dst])
            pltpu.sync_copy(r_gi, o_gi.at[dst])
            pltpu.sync_copy(r_mu, o_mu.at[dst])
            pltpu.sync_copy(r_mi, o_mi.at[dst])

    return _sc_gather4


BLK = 2048


def _tc_body(gu, gi, mu, mi, w1a, w1b, b1, w2, b2, w3, b3, wog, wom, bo, out):
    h = jnp.dot(mu[...], w1a[...], preferred_element_type=jnp.float32)
    h = h + jnp.dot(mi[...], w1b[...], preferred_element_type=jnp.float32)
    h = jnp.maximum(h + b1[...], 0.0)
    h = jnp.maximum(
        jnp.dot(h, w2[...], preferred_element_type=jnp.float32) + b2[...], 0.0)
    pm = jnp.dot(h, w3[...], preferred_element_type=jnp.float32) + b3[...]
    pg = gu[...] * gi[...]
    logit = (jnp.sum(pg * wog[...], axis=1, keepdims=True)
             + jnp.sum(pm * wom[...], axis=1, keepdims=True) + bo[...])
    out[...] = jax.nn.sigmoid(logit)


def _tc_dense(gu, gi, mu, mi, w1a, w1b, b1, w2, b2, w3, b3, wog, wom, bo):
    full = lambda shape: pl.BlockSpec(shape, lambda i: (0, 0))
    return pl.pallas_call(
        _tc_body,
        grid=(B // BLK,),
        in_specs=[
            pl.BlockSpec((BLK, E), lambda i: (i, 0)),
            pl.BlockSpec((BLK, E), lambda i: (i, 0)),
            pl.BlockSpec((BLK, E), lambda i: (i, 0)),
            pl.BlockSpec((BLK, E), lambda i: (i, 0)),
            full((E, H1)), full((E, H1)), full((1, H1)),
            full((H1, H2)), full((1, H2)),
            full((H2, H3)), full((1, H3)),
            full((1, E)), full((1, H3)), full((1, 1)),
        ],
        out_specs=pl.BlockSpec((BLK, 1), lambda i: (i, 0)),
        out_shape=jax.ShapeDtypeStruct((B, 1), jnp.float32),
    )(gu, gi, mu, mi, w1a, w1b, b1, w2, b2, w3, b3, wog, wom, bo)


def kernel(user, item, gmf_user_w, gmf_item_w, mlp_user_w, mlp_item_w,
           W1, b1, W2, b2, W3, b3, Wo, bo):
    gu, gi, mu, mi = _build_sc_gather4()(user, item, gmf_user_w, gmf_item_w,
                                         mlp_user_w, mlp_item_w)
    w1a, w1b = W1[:E], W1[E:]
    wog = Wo[:E].reshape(1, E)
    wom = Wo[E:].reshape(1, H3)
    return _tc_dense(gu, gi, mu, mi, w1a, w1b,
                     b1.reshape(1, H1), W2, b2.reshape(1, H2),
                     W3, b3.reshape(1, H3), wog, wom, bo.reshape(1, 1))
